# Initial kernel scaffold; baseline (speedup 1.0000x reference)
import jax
import jax.numpy as jnp
from jax.experimental import pallas as pl


def kernel(indices, table):
    raise NotImplementedError("write your pallas kernel here")


# speedup vs baseline: 2.9751x; 2.9751x over previous
import functools

import jax
import jax.numpy as jnp
from jax import lax
from jax.experimental import pallas as pl
from jax.experimental.pallas import tpu as pltpu
from jax.experimental.pallas import tpu_sc as plsc

NC = 2
NS = 16
NW = NC * NS


@functools.lru_cache(maxsize=None)
def _build(n, d):
    per_w = n // NW
    chunk = 128
    n_chunks = per_w // chunk
    mesh = plsc.VectorSubcoreMesh(core_axis_name="c", subcore_axis_name="s")

    @functools.partial(
        pl.kernel,
        out_type=jax.ShapeDtypeStruct((n, d), jnp.float32),
        mesh=mesh,
        scratch_types=[
            pltpu.VMEM((per_w,), jnp.int32),
            pltpu.VMEM((chunk, d), jnp.float32),
            pltpu.SemaphoreType.DMA,
        ],
    )
    def gather_kernel(idx_hbm, table_hbm, out_hbm, idx_v, rows_v, gsem):
        wid = lax.axis_index("s") * NC + lax.axis_index("c")
        base = wid * per_w
        pltpu.sync_copy(idx_hbm.at[pl.ds(base, per_w)], idx_v)

        def chunk_body(i, _):
            off = i * chunk
            pltpu.async_copy(
                table_hbm.at[idx_v.at[pl.ds(off, chunk)]], rows_v, gsem
            ).wait()
            pltpu.sync_copy(rows_v, out_hbm.at[pl.ds(base + off, chunk)])
            return 0

        lax.fori_loop(0, n_chunks, chunk_body, 0)

    return gather_kernel


def kernel(indices, table):
    b, l = indices.shape
    v, d = table.shape
    idx_flat = indices.reshape(-1).astype(jnp.int32)
    out = _build(b * l, d)(idx_flat, table)
    return out.reshape(b, l, d)


# speedup vs baseline: 3.3289x; 1.1189x over previous
import functools

import jax
import jax.numpy as jnp
from jax import lax
from jax.experimental import pallas as pl
from jax.experimental.pallas import tpu as pltpu
from jax.experimental.pallas import tpu_sc as plsc

NC = 2
NS = 16
NW = NC * NS


@functools.lru_cache(maxsize=None)
def _build(n, d):
    per_w = n // NW
    chunk = 80
    nbuf = 4
    n_chunks = per_w // chunk
    n_groups = n_chunks // nbuf
    mesh = plsc.VectorSubcoreMesh(core_axis_name="c", subcore_axis_name="s")

    @functools.partial(
        pl.kernel,
        out_type=jax.ShapeDtypeStruct((n, d), jnp.float32),
        mesh=mesh,
        scratch_types=[
            pltpu.VMEM((per_w,), jnp.int32),
            [pltpu.VMEM((chunk, d), jnp.float32) for _ in range(nbuf)],
            [pltpu.SemaphoreType.DMA for _ in range(nbuf)],
            [pltpu.SemaphoreType.DMA for _ in range(nbuf)],
        ],
    )
    def gather_kernel(idx_hbm, table_hbm, out_hbm, idx_v, bufs, gsems, wsems):
        wid = lax.axis_index("s") * NC + lax.axis_index("c")
        base = wid * per_w
        pltpu.sync_copy(idx_hbm.at[pl.ds(base, per_w)], idx_v)

        def fire_gather(i, b):
            pltpu.async_copy(
                table_hbm.at[idx_v.at[pl.ds(i * chunk, chunk)]],
                bufs[b], gsems[b],
            )

        def fire_write(i, b):
            pltpu.async_copy(
                bufs[b], out_hbm.at[pl.ds(base + i * chunk, chunk)], wsems[b]
            )

        def drain(sem, buf):
            pltpu.make_async_copy(out_hbm.at[pl.ds(0, chunk)], buf, sem).wait()

        for b in range(nbuf):
            fire_gather(b, b)

        def group_body(g, _):
            i0 = g * nbuf
            for b in range(nbuf):
                drain(gsems[b], bufs[b])
                fire_write(i0 + b, b)
            for b in range(nbuf):
                drain(wsems[b], bufs[b])
                fire_gather(i0 + nbuf + b, b)
            return 0

        lax.fori_loop(0, n_groups - 1, group_body, 0)

        i0 = (n_groups - 1) * nbuf
        for b in range(nbuf):
            drain(gsems[b], bufs[b])
            fire_write(i0 + b, b)
        for b in range(nbuf):
            drain(wsems[b], bufs[b])

    return gather_kernel


def kernel(indices, table):
    b, l = indices.shape
    v, d = table.shape
    idx_flat = indices.reshape(-1).astype(jnp.int32)
    out = _build(b * l, d)(idx_flat, table)
    return out.reshape(b, l, d)


# speedup vs baseline: 5.7379x; 1.7236x over previous
import functools

import jax
import jax.numpy as jnp
from jax import lax
from jax.experimental import pallas as pl
from jax.experimental.pallas import tpu as pltpu
from jax.experimental.pallas import tpu_sc as plsc

NC = 2
NS = 16
NW = NC * NS
LP = 56


@functools.lru_cache(maxsize=None)
def _build(b, l, d):
    rows_w = b // NW
    rb = 4
    nbuf = 2
    n_chunks = rows_w // rb
    idx_w = rows_w * LP
    mesh = plsc.VectorSubcoreMesh(core_axis_name="c", subcore_axis_name="s")

    @functools.partial(
        pl.kernel,
        out_type=jax.ShapeDtypeStruct((b, l, d), jnp.float32),
        mesh=mesh,
        compiler_params=pltpu.CompilerParams(use_tc_tiling_on_sc=True),
        scratch_types=[
            pltpu.VMEM((idx_w,), jnp.int32),
            [pltpu.VMEM((rb, l, d), jnp.float32) for _ in range(nbuf)],
            [pltpu.SemaphoreType.DMA for _ in range(nbuf)],
            [pltpu.SemaphoreType.DMA for _ in range(nbuf)],
        ],
    )
    def gather_kernel(idx_hbm, table_hbm, out_hbm, idx_v, bufs, gsems, wsems):
        wid = lax.axis_index("s") * NC + lax.axis_index("c")
        row0 = wid * rows_w
        pltpu.sync_copy(idx_hbm.at[pl.ds(wid * idx_w, idx_w)], idx_v)

        def fire_gathers(c, buf_i):
            for r in range(rb):
                pltpu.async_copy(
                    table_hbm.at[idx_v.at[pl.ds((c * rb + r) * LP, l)]],
                    bufs[buf_i].at[r],
                    gsems[buf_i],
                )

        def fire_write(c, buf_i):
            pltpu.async_copy(
                bufs[buf_i], out_hbm.at[pl.ds(row0 + c * rb, rb)],
                wsems[buf_i],
            )

        def drain(sem):
            pltpu.make_async_copy(out_hbm.at[pl.ds(0, rb)], bufs[0], sem).wait()

        for bi in range(nbuf):
            fire_gathers(bi, bi)

        def group_body(g, _):
            c0 = g * nbuf
            for bi in range(nbuf):
                drain(gsems[bi])
                fire_write(c0 + bi, bi)
            for bi in range(nbuf):
                drain(wsems[bi])
                fire_gathers(c0 + nbuf + bi, bi)
            return 0

        lax.fori_loop(0, n_chunks // nbuf - 1, group_body, 0)

        c0 = n_chunks - nbuf
        for bi in range(nbuf):
            drain(gsems[bi])
            fire_write(c0 + bi, bi)
        for bi in range(nbuf):
            drain(wsems[bi])

    return gather_kernel


def kernel(indices, table):
    b, l = indices.shape
    v, d = table.shape
    idx_pad = jnp.pad(indices.astype(jnp.int32), ((0, 0), (0, LP - l)))
    return _build(b, l, d)(idx_pad.reshape(-1), table)


# speedup vs baseline: 10.1670x; 1.7719x over previous
import functools

import jax
import jax.numpy as jnp
from jax import lax
from jax.experimental import pallas as pl
from jax.experimental.pallas import tpu as pltpu
from jax.experimental.pallas import tpu_sc as plsc

NC = 2
NS = 16
NW = NC * NS


@functools.lru_cache(maxsize=None)
def _build(n, d):
    per_w = n // NW
    chunk = 80
    nbuf = 4
    n_chunks = per_w // chunk
    n_groups = n_chunks // nbuf
    mesh = plsc.VectorSubcoreMesh(core_axis_name="c", subcore_axis_name="s")

    @functools.partial(
        pl.kernel,
        out_type=jax.ShapeDtypeStruct((n, d), jnp.float32),
        mesh=mesh,
        scratch_types=[
            pltpu.VMEM((per_w,), jnp.int32),
            [pltpu.VMEM((chunk, d), jnp.float32) for _ in range(nbuf)],
            [pltpu.SemaphoreType.DMA for _ in range(nbuf)],
            [pltpu.SemaphoreType.DMA for _ in range(nbuf)],
        ],
    )
    def gather_kernel(idx_hbm, table_hbm, out_hbm, idx_v, bufs, gsems, wsems):
        wid = lax.axis_index("s") * NC + lax.axis_index("c")
        base = wid * per_w
        pltpu.sync_copy(idx_hbm.at[pl.ds(base, per_w)], idx_v)

        def fire_gather(i, b):
            pltpu.async_copy(
                table_hbm.at[idx_v.at[pl.ds(i * chunk, chunk)]],
                bufs[b], gsems[b],
            )

        def fire_write(i, b):
            pltpu.async_copy(
                bufs[b], out_hbm.at[pl.ds(base + i * chunk, chunk)], wsems[b]
            )

        def drain(sem, buf):
            pltpu.make_async_copy(out_hbm.at[pl.ds(0, chunk)], buf, sem).wait()

        for b in range(nbuf):
            fire_gather(b, b)

        def group_body(g, _):
            i0 = g * nbuf
            for b in range(nbuf):
                drain(gsems[b], bufs[b])
                fire_write(i0 + b, b)
            for b in range(nbuf):
                drain(wsems[b], bufs[b])
                fire_gather(i0 + nbuf + b, b)
            return 0

        lax.fori_loop(0, n_groups - 1, group_body, 0)

        i0 = (n_groups - 1) * nbuf
        for b in range(nbuf):
            drain(gsems[b], bufs[b])
            fire_write(i0 + b, b)
        for b in range(nbuf):
            drain(wsems[b], bufs[b])

    return gather_kernel


def kernel(indices, table):
    b, l = indices.shape
    v, d = table.shape
    idx_t = jnp.transpose(indices.astype(jnp.int32)).reshape(-1)
    out = _build(b * l, d)(idx_t, table)
    return jnp.transpose(out.reshape(l, b, d), (1, 0, 2))
